# Initial kernel scaffold; baseline (speedup 1.0000x reference)
import jax
import jax.numpy as jnp
from jax.experimental import pallas as pl


def kernel(user_embed, item_embed, edge_vals, edge_index, users, pos_items, neg_items):
    raise NotImplementedError("write your pallas kernel here")


# speedup vs baseline: 2.4813x; 2.4813x over previous
import functools

import jax
import jax.numpy as jnp
from jax import lax
from jax.experimental import pallas as pl
from jax.experimental.pallas import tpu as pltpu
from jax.experimental.pallas import tpu_sc as plsc

N_USERS = 25000
N_ITEMS = 25000
N = N_USERS + N_ITEMS
D = 64
E = 800000
HOPS = 3
K = 4
B = 4096
DECAY = 1e-4

NC = 2
NS = 16
L = 16

HALF = N // NC
CHUNK = 2048
GBLK = 128
CHUNKS_PER_SUB = 25
EPS = CHUNK * CHUNKS_PER_SUB
E_PAD = EPS * NS
EBLKS = E_PAD // GBLK
BLKS_PER_SUB = EBLKS // NS
ACC_ROWS = 26624


def _spmm_body(rows_hbm, cols_hbm, vals_hbm, agg_hbm, out_hbm,
               rows_v, cols_v, vals_v, sidx_v, gbuf, zbuf, acc):
    c = lax.axis_index("c")
    s = lax.axis_index("s")
    lo = c * HALF
    hi = lo + HALF
    trash16 = (HALF + s * L + lax.iota(jnp.int32, L)).astype(jnp.int32)

    zero16 = jnp.zeros((L,), jnp.float32)

    def zrow(r, carry):
        for k2 in range(4):
            zbuf[r, pl.ds(k2 * L, L)] = zero16
        return carry

    lax.fori_loop(0, GBLK, zrow, 0)

    def zblk(b2, carry):
        pltpu.sync_copy(zbuf, acc.at[pl.ds((s * 13 + b2) * 128, 128)])
        return carry

    lax.fori_loop(0, 13, zblk, 0)
    plsc.subcore_barrier()

    def chunk_body(i, carry):
        blk0 = (s * CHUNKS_PER_SUB + i) * (CHUNK // GBLK)
        nblk = CHUNK // GBLK
        pltpu.sync_copy(rows_hbm.at[pl.ds(blk0, nblk)], rows_v)
        pltpu.sync_copy(cols_hbm.at[pl.ds(blk0, nblk)], cols_v)
        pltpu.sync_copy(vals_hbm.at[pl.ds(blk0, nblk)], vals_v)

        def prep(j, carry2):
            for t in range(8):
                r = rows_v[j, pl.ds(t * L, L)]
                ok = (r >= lo) & (r < hi)
                sidx_v[j, pl.ds(t * L, L)] = jnp.where(ok, r - lo, trash16)
            return carry2

        lax.fori_loop(0, nblk, prep, 0)

        def jloop(j, carry2):
            pltpu.sync_copy(agg_hbm.at[cols_v.at[j]], gbuf)

            def scale(g, carry3):
                val16 = vals_v[j, pl.ds(g * L, L)]
                for t in range(L):
                    v = val16[jnp.full((L,), t, jnp.int32)]
                    e = g * L + t
                    for k2 in range(4):
                        gbuf[e, pl.ds(k2 * L, L)] = (
                            gbuf[e, pl.ds(k2 * L, L)] * v)
                return carry3

            lax.fori_loop(0, GBLK // L, scale, 0)
            pltpu.sync_copy(gbuf, acc.at[sidx_v.at[j]], add=True)
            return carry2

        lax.fori_loop(0, nblk, jloop, 0)
        return carry

    lax.fori_loop(0, CHUNKS_PER_SUB, chunk_body, 0)
    plsc.subcore_barrier()

    def wb(b2, carry):
        cid = b2 * NS + s

        @pl.when(cid < 25)
        def _():
            pltpu.sync_copy(acc.at[pl.ds(cid * 1000, 1000)],
                            out_hbm.at[pl.ds(lo + cid * 1000, 1000)])
        return carry

    lax.fori_loop(0, 2, wb, 0)


def _sc_spmm(rows2, cols2, vals2, agg):
    mesh = plsc.VectorSubcoreMesh(core_axis_name="c", subcore_axis_name="s")
    fn = pl.kernel(
        _spmm_body,
        out_type=jax.ShapeDtypeStruct((N, D), jnp.float32),
        mesh=mesh,
        compiler_params=pltpu.CompilerParams(use_tc_tiling_on_sc=False),
        scratch_types=[
            pltpu.VMEM((CHUNK // GBLK, GBLK), jnp.int32),
            pltpu.VMEM((CHUNK // GBLK, GBLK), jnp.int32),
            pltpu.VMEM((CHUNK // GBLK, GBLK), jnp.float32),
            pltpu.VMEM((CHUNK // GBLK, GBLK), jnp.int32),
            pltpu.VMEM((GBLK, D), jnp.float32),
            pltpu.VMEM((GBLK, D), jnp.float32),
            pltpu.VMEM_SHARED((ACC_ROWS, D), jnp.float32),
        ],
    )
    return fn(rows2, cols2, vals2, agg)


def _pool_body(e0, a1, a2, a3, u_i, p_i, n_i,
               ue_o, pe_o, ne_o, u0_o, p0_o, n0_o,
               idx_v, grow, accb):
    c = lax.axis_index("c")
    s = lax.axis_index("s")
    w = s * NC + c

    def seg(idx_hbm, nblk, pooled_o, hop0_o, seg_rows):
        base = w * seg_rows

        def blk(bi, carry):
            off = base + bi * GBLK
            pltpu.sync_copy(idx_hbm.at[pl.ds(off, GBLK)], idx_v)
            pltpu.sync_copy(e0.at[idx_v], grow)
            pltpu.sync_copy(grow, hop0_o.at[pl.ds(off, GBLK)])

            def cp(r, carry2):
                for k2 in range(4):
                    accb[r, pl.ds(k2 * L, L)] = grow[r, pl.ds(k2 * L, L)]
                return carry2

            lax.fori_loop(0, GBLK, cp, 0)
            for tbl in (a1, a2, a3):
                pltpu.sync_copy(tbl.at[idx_v], grow)

                def add(r, carry2):
                    for k2 in range(4):
                        accb[r, pl.ds(k2 * L, L)] = (
                            accb[r, pl.ds(k2 * L, L)]
                            + grow[r, pl.ds(k2 * L, L)])
                    return carry2

                lax.fori_loop(0, GBLK, add, 0)
            pltpu.sync_copy(accb, pooled_o.at[pl.ds(off, GBLK)])
            return carry

        lax.fori_loop(0, nblk, blk, 0)

    seg(u_i, 1, ue_o, u0_o, B // (NC * NS))
    seg(p_i, 1, pe_o, p0_o, B // (NC * NS))
    seg(n_i, 4, ne_o, n0_o, (B * K) // (NC * NS))


def _sc_pool(e0, a1, a2, a3, u_i, p_i, n_i):
    mesh = plsc.VectorSubcoreMesh(core_axis_name="c", subcore_axis_name="s")
    f32 = jnp.float32
    fn = pl.kernel(
        _pool_body,
        out_type=(
            jax.ShapeDtypeStruct((B, D), f32),
            jax.ShapeDtypeStruct((B, D), f32),
            jax.ShapeDtypeStruct((B * K, D), f32),
            jax.ShapeDtypeStruct((B, D), f32),
            jax.ShapeDtypeStruct((B, D), f32),
            jax.ShapeDtypeStruct((B * K, D), f32),
        ),
        mesh=mesh,
        compiler_params=pltpu.CompilerParams(use_tc_tiling_on_sc=False),
        scratch_types=[
            pltpu.VMEM((GBLK,), jnp.int32),
            pltpu.VMEM((GBLK, D), f32),
            pltpu.VMEM((GBLK, D), f32),
        ],
    )
    return fn(e0, a1, a2, a3, u_i, p_i, n_i)


def _loss_body(ue, pe, ne, u0, p0, n0, out_ref):
    u = ue[...]
    p = pe[...]
    scale = 1.0 / ((HOPS + 1) * (HOPS + 1))

    def softplus(x):
        return jnp.maximum(x, 0.0) + jnp.log1p(jnp.exp(-jnp.abs(x)))

    ps = jnp.sum(u * p, axis=1, keepdims=True) * scale
    tot = softplus(-ps)
    for k in range(K):
        nk = ne[pl.ds(k * B, B), :]
        nsc = jnp.sum(u * nk, axis=1, keepdims=True) * scale
        tot = tot + softplus(nsc)
    mf = jnp.sum(tot) / B
    reg = (jnp.sum(u0[...] ** 2) + jnp.sum(p0[...] ** 2)
           + jnp.sum(n0[...] ** 2)) * 0.5
    out_ref[0, 0] = mf + DECAY * reg / B


def _tc_loss(ue, pe, ne, u0, p0, n0):
    return pl.pallas_call(
        _loss_body,
        out_shape=jax.ShapeDtypeStruct((1, 1), jnp.float32),
        out_specs=pl.BlockSpec(memory_space=pltpu.SMEM),
    )(ue, pe, ne, u0, p0, n0)


def kernel(user_embed, item_embed, edge_vals, edge_index, users, pos_items,
           neg_items):
    all_embed = jnp.concatenate([user_embed, item_embed], axis=0)
    rows = edge_index[0]
    cols = edge_index[1]
    pad = E_PAD - E
    rows_p = jnp.concatenate([rows, jnp.full((pad,), N, jnp.int32)])
    cols_p = jnp.concatenate(
        [cols, (jnp.arange(pad, dtype=jnp.int32) * 997) % N])
    vals_p = jnp.concatenate([edge_vals, jnp.zeros((pad,), jnp.float32)])
    rows2 = rows_p.reshape(EBLKS, GBLK)
    cols2 = cols_p.reshape(EBLKS, GBLK)
    vals2 = vals_p.reshape(EBLKS, GBLK)

    a1 = _sc_spmm(rows2, cols2, vals2, all_embed)
    a2 = _sc_spmm(rows2, cols2, vals2, a1)
    a3 = _sc_spmm(rows2, cols2, vals2, a2)

    u_i = users
    p_i = pos_items + N_USERS
    n_i = neg_items.T.reshape(-1) + N_USERS

    ue, pe, ne, u0, p0, n0 = _sc_pool(all_embed, a1, a2, a3, u_i, p_i, n_i)
    out = _tc_loss(ue, pe, ne, u0, p0, n0)
    return out[0, 0]
